# baseline (speedup 1.0000x reference)
import jax
import jax.numpy as jnp
from jax import lax
from jax.experimental import pallas as pl
from jax.experimental.pallas import tpu as pltpu
from jax.experimental.pallas import tpu_sc as plsc

N = 10000
E = 320000
D_IN = 128
H = 256
DH = H // 2

NC = 2
NS = 16
CHUNK = 128
N_CHUNKS = E // CHUNK
CHUNKS_PER_TILE = -(-N_CHUNKS // NS)
ZROWS = 125
N_ZSPANS = N // ZROWS
OUT_SPAN = 640
OUT_SPAN_LAST = N - 15 * OUT_SPAN
DEG_OUT_ROWS = 1256

_sc_mesh = plsc.VectorSubcoreMesh(core_axis_name="c", subcore_axis_name="s")


def _fill(buf, nrows, ncols, val):
    @pl.loop(0, nrows)
    def _(r):
        @pl.loop(0, ncols // 16)
        def _(c):
            buf[r, pl.ds(c * 16, 16)] = jnp.full((16,), val, jnp.float32)


def _zero_fill(buf, nrows, ncols):
    _fill(buf, nrows, ncols, 0.0)


def _sc_deg_body(src_hbm, dst_hbm, deg_hbm, idx0_v, idx1_v, ones_v, zeros_v,
                 sem0, sem1, deg_sh):
    cid = lax.axis_index("c")
    sid = lax.axis_index("s")

    _fill(ones_v, CHUNK, 128, 1.0)
    _zero_fill(zeros_v, ZROWS, 128)

    @pl.loop(0, N_ZSPANS)
    def _(k):
        @pl.when(lax.rem(k, NS) == sid)
        def _():
            pltpu.sync_copy(zeros_v, deg_sh.at[pl.ds(k * ZROWS, ZROWS)])
    plsc.subcore_barrier()

    def count(e_hbm):
        idx = (idx0_v, idx1_v)
        sems = (sem0, sem1)

        def start(c, b):
            pltpu.sync_copy(e_hbm.at[pl.ds(c * CHUNK, CHUNK)], idx[b])
            pltpu.async_copy(ones_v, deg_sh.at[idx[b]], sems[b], add=True)

        def drain(b):
            pltpu.make_async_copy(ones_v, deg_sh.at[idx[b]], sems[b]).wait()

        start(sid, 0)

        @pl.loop(0, CHUNKS_PER_TILE, step=2)
        def _(k):
            c_a = k * NS + sid
            c_b = c_a + NS
            c_c = c_b + NS

            @pl.when(c_b < N_CHUNKS)
            def _():
                start(c_b, 1)

            @pl.when(c_a < N_CHUNKS)
            def _():
                drain(0)

            @pl.when(c_c < N_CHUNKS)
            def _():
                start(c_c, 0)

            @pl.when(c_b < N_CHUNKS)
            def _():
                drain(1)

    @pl.when(cid == 0)
    def _():
        count(src_hbm)

    @pl.when(cid == 1)
    def _():
        count(dst_hbm)

    plsc.subcore_barrier()
    rbase = sid * OUT_SPAN

    @pl.when(sid < NS - 1)
    def _():
        pltpu.sync_copy(deg_sh.at[pl.ds(rbase, OUT_SPAN)],
                        deg_hbm.at[cid, pl.ds(rbase, OUT_SPAN)])

    @pl.when(sid == NS - 1)
    def _():
        pltpu.sync_copy(deg_sh.at[pl.ds(rbase, OUT_SPAN_LAST)],
                        deg_hbm.at[cid, pl.ds(rbase, OUT_SPAN_LAST)])


_sc_deg = pl.kernel(
    _sc_deg_body,
    out_type=jax.ShapeDtypeStruct((NC, N, 128), jnp.float32),
    mesh=_sc_mesh,
    scratch_types=[
        pltpu.VMEM((CHUNK,), jnp.int32),
        pltpu.VMEM((CHUNK,), jnp.int32),
        pltpu.VMEM((CHUNK, 128), jnp.float32),
        pltpu.VMEM((ZROWS, 128), jnp.float32),
        pltpu.SemaphoreType.DMA,
        pltpu.SemaphoreType.DMA,
        pltpu.VMEM_SHARED((N, 128), jnp.float32),
    ],
)


def _zero_spmem_acc(zrows, agg_sh, sid):
    _zero_fill(zrows, ZROWS, DH)

    @pl.loop(0, N_ZSPANS)
    def _(k):
        @pl.when(lax.rem(k, NS) == sid)
        def _():
            pltpu.sync_copy(zrows.at[pl.ds(0, ZROWS)],
                            agg_sh.at[pl.ds(k * ZROWS, ZROWS)])
    plsc.subcore_barrier()


IDXB = 8
CHUNKS_PAD = 2560
E_PAD = CHUNKS_PAD * CHUNK


def _edge_pipeline_blk(srcb, dstb, h_hbm, agg_sh, bufs, t0, nblk):
    sblk, dblk, rows, gsem, isem = bufs

    def pf(q, p):
        base = t0 + q * IDXB
        pltpu.async_copy(srcb.at[pl.ds(base, IDXB)], sblk[p], isem[p])
        pltpu.async_copy(dstb.at[pl.ds(base, IDXB)], dblk[p], isem[p])

    def pf_wait(q, p):
        base = t0 + q * IDXB
        pltpu.make_async_copy(srcb.at[pl.ds(base, IDXB)], sblk[p],
                              isem[p]).wait()
        pltpu.make_async_copy(dstb.at[pl.ds(base, IDXB)], dblk[p],
                              isem[p]).wait()

    def fire(j, p, rb):
        pltpu.async_copy(h_hbm.at[sblk[p].at[j]], rows[rb], gsem[rb])

    def proc(j, p, rb):
        pltpu.make_async_copy(h_hbm.at[sblk[p].at[j]], rows[rb],
                              gsem[rb]).wait()
        pltpu.sync_copy(rows[rb], agg_sh.at[dblk[p].at[j]], add=True)

    pf(0, 0)

    @pl.loop(0, nblk, step=2)
    def _(k):
        for s in range(2):
            p = s
            po = 1 - s
            pf_wait(k + s, p)
            for j in range(IDXB):
                rb = j % 2
                fire(j, p, rb)
                if j == 0:
                    if s == 0:
                        @pl.when(k >= 1)
                        def _():
                            proc(IDXB - 1, po, 1)
                    else:
                        proc(IDXB - 1, po, 1)

                    @pl.when(k + s + 1 < nblk)
                    def _():
                        pf(k + s + 1, po)
                else:
                    proc(j - 1, p, (j - 1) % 2)

    proc(IDXB - 1, (nblk - 1) % 2, 1)


def _acc_copy_out(agg_sh, agg_hbm, cid, sid):
    plsc.subcore_barrier()
    rbase = sid * OUT_SPAN

    @pl.when(sid < NS - 1)
    def _():
        pltpu.sync_copy(agg_sh.at[pl.ds(rbase, OUT_SPAN)],
                        agg_hbm.at[cid, pl.ds(rbase, OUT_SPAN)])

    @pl.when(sid == NS - 1)
    def _():
        pltpu.sync_copy(agg_sh.at[pl.ds(rbase, OUT_SPAN_LAST)],
                        agg_hbm.at[cid, pl.ds(rbase, OUT_SPAN_LAST)])


TILE_BLKS = CHUNKS_PAD // NS // IDXB
TILE_BLKS_H = CHUNKS_PAD // NC // NS // IDXB


def _sc_agg_body(srcb, dstb, ha_hbm, hb_hbm, agg_hbm,
                 sblk0, sblk1, dblk0, dblk1, rows0, rows1,
                 gsem0, gsem1, isem0, isem1, agg_sh):
    cid = lax.axis_index("c")
    sid = lax.axis_index("s")
    _zero_spmem_acc(rows0, agg_sh, sid)

    bufs = ((sblk0, sblk1), (dblk0, dblk1), (rows0, rows1),
            (gsem0, gsem1), (isem0, isem1))

    @pl.when(cid == 0)
    def _():
        _edge_pipeline_blk(srcb, dstb, ha_hbm, agg_sh, bufs,
                           sid * (TILE_BLKS * IDXB), TILE_BLKS)

    @pl.when(cid == 1)
    def _():
        _edge_pipeline_blk(srcb, dstb, hb_hbm, agg_sh, bufs,
                           sid * (TILE_BLKS * IDXB), TILE_BLKS)

    _acc_copy_out(agg_sh, agg_hbm, cid, sid)


_AGG_SCRATCH = (
    [pltpu.VMEM((IDXB, CHUNK), jnp.int32)] * 4
    + [pltpu.VMEM((CHUNK, DH), jnp.float32)] * 2
    + [pltpu.SemaphoreType.DMA] * 4
    + [pltpu.VMEM_SHARED((N + IDXB, DH), jnp.float32)]
)

_sc_agg = pl.kernel(
    _sc_agg_body,
    out_type=jax.ShapeDtypeStruct((NC, N, DH), jnp.float32),
    mesh=_sc_mesh,
    scratch_types=list(_AGG_SCRATCH),
)

HALF_CHUNKS = N_CHUNKS // NC
CHUNKS_PER_TILE_H = -(-HALF_CHUNKS // NS)


def _sc_agg1_body(srcb, dstb, xs_hbm, agg_hbm,
                  sblk0, sblk1, dblk0, dblk1, rows0, rows1,
                  gsem0, gsem1, isem0, isem1, agg_sh):
    cid = lax.axis_index("c")
    sid = lax.axis_index("s")
    _zero_spmem_acc(rows0, agg_sh, sid)

    bufs = ((sblk0, sblk1), (dblk0, dblk1), (rows0, rows1),
            (gsem0, gsem1), (isem0, isem1))
    t0 = (cid * NS + sid) * (TILE_BLKS_H * IDXB)
    _edge_pipeline_blk(srcb, dstb, xs_hbm, agg_sh, bufs, t0, TILE_BLKS_H)

    _acc_copy_out(agg_sh, agg_hbm, cid, sid)


_sc_agg1 = pl.kernel(
    _sc_agg1_body,
    out_type=jax.ShapeDtypeStruct((NC, N, DH), jnp.float32),
    mesh=_sc_mesh,
    scratch_types=list(_AGG_SCRATCH),
)


def _dot(a, b):
    return jnp.dot(a, b, preferred_element_type=jnp.float32,
                   precision=lax.Precision.HIGHEST)


BR = 2000
G_TC = N // BR


def _tc_prep_body(x_ref, dsrc_ref, ddst_ref, xs_ref, nsrc_ref, ndst_ref):
    nsrc = lax.rsqrt(jnp.maximum(dsrc_ref[...], 1.0))
    ndst = lax.rsqrt(jnp.maximum(ddst_ref[...], 1.0))
    nsrc_ref[...] = nsrc
    ndst_ref[...] = ndst
    xs_ref[...] = x_ref[...] * nsrc


_tc_prep = pl.pallas_call(
    _tc_prep_body,
    grid=(G_TC,),
    in_specs=[
        pl.BlockSpec((BR, D_IN), lambda i: (i, 0)),
        pl.BlockSpec((BR, 1), lambda i: (i, 0)),
        pl.BlockSpec((BR, 1), lambda i: (i, 0)),
    ],
    out_specs=(
        pl.BlockSpec((BR, D_IN), lambda i: (i, 0)),
        pl.BlockSpec((BR, 1), lambda i: (i, 0)),
        pl.BlockSpec((BR, 1), lambda i: (i, 0)),
    ),
    out_shape=(
        jax.ShapeDtypeStruct((N, D_IN), jnp.float32),
        jax.ShapeDtypeStruct((N, 1), jnp.float32),
        jax.ShapeDtypeStruct((N, 1), jnp.float32),
    ),
)


def _tc_l1_body(agg_ref, w1_ref, b1_ref, ndst_ref, nsrc_ref, w2_ref,
                ha_ref, hb_ref, r_ref):
    a1 = agg_ref[0] + agg_ref[1]
    g = jnp.maximum(_dot(a1, w1_ref[...]) * ndst_ref[...] + b1_ref[...], 0.0)

    @pl.when(pl.program_id(0) == 0)
    def _():
        r_ref[...] = jnp.zeros((1, H), jnp.float32)

    r_ref[...] += jnp.sum(g, axis=0, keepdims=True) * (1.0 / N)
    hn = _dot(g, w2_ref[...]) * nsrc_ref[...]
    ha_ref[...] = hn[:, :DH]
    hb_ref[...] = hn[:, DH:]


_tc_l1 = pl.pallas_call(
    _tc_l1_body,
    grid=(G_TC,),
    in_specs=[
        pl.BlockSpec((NC, BR, DH), lambda i: (0, i, 0)),
        pl.BlockSpec((D_IN, H), lambda i: (0, 0)),
        pl.BlockSpec((1, H), lambda i: (0, 0)),
        pl.BlockSpec((BR, 1), lambda i: (i, 0)),
        pl.BlockSpec((BR, 1), lambda i: (i, 0)),
        pl.BlockSpec((H, H), lambda i: (0, 0)),
    ],
    out_specs=(
        pl.BlockSpec((BR, DH), lambda i: (i, 0)),
        pl.BlockSpec((BR, DH), lambda i: (i, 0)),
        pl.BlockSpec((1, H), lambda i: (0, 0)),
    ),
    out_shape=(
        jax.ShapeDtypeStruct((N, DH), jnp.float32),
        jax.ShapeDtypeStruct((N, DH), jnp.float32),
        jax.ShapeDtypeStruct((1, H), jnp.float32),
    ),
)


def _gcn_epilogue(agg_ref, ndst_ref, b_ref):
    ndst = ndst_ref[...]
    ga = jnp.maximum(agg_ref[0] * ndst + b_ref[0:1, :DH], 0.0)
    gb = jnp.maximum(agg_ref[1] * ndst + b_ref[0:1, DH:], 0.0)
    return ga, gb


def _tc_mid_body(agg_ref, ndst_ref, b_ref, w_ref, nsrc_ref,
                 ha_ref, hb_ref, r_ref):
    ga, gb = _gcn_epilogue(agg_ref, ndst_ref, b_ref)
    ra = jnp.sum(ga, axis=0, keepdims=True) * (1.0 / N)
    rb = jnp.sum(gb, axis=0, keepdims=True) * (1.0 / N)

    @pl.when(pl.program_id(0) == 0)
    def _():
        r_ref[...] = jnp.zeros((1, H), jnp.float32)

    r_ref[0:1, :DH] += ra
    r_ref[0:1, DH:] += rb
    hn = (_dot(ga, w_ref[:DH, :]) + _dot(gb, w_ref[DH:, :])) * nsrc_ref[...]
    ha_ref[...] = hn[:, :DH]
    hb_ref[...] = hn[:, DH:]


_tc_mid = pl.pallas_call(
    _tc_mid_body,
    grid=(G_TC,),
    in_specs=[
        pl.BlockSpec((NC, BR, DH), lambda i: (0, i, 0)),
        pl.BlockSpec((BR, 1), lambda i: (i, 0)),
        pl.BlockSpec((1, H), lambda i: (0, 0)),
        pl.BlockSpec((H, H), lambda i: (0, 0)),
        pl.BlockSpec((BR, 1), lambda i: (i, 0)),
    ],
    out_specs=(
        pl.BlockSpec((BR, DH), lambda i: (i, 0)),
        pl.BlockSpec((BR, DH), lambda i: (i, 0)),
        pl.BlockSpec((1, H), lambda i: (0, 0)),
    ),
    out_shape=(
        jax.ShapeDtypeStruct((N, DH), jnp.float32),
        jax.ShapeDtypeStruct((N, DH), jnp.float32),
        jax.ShapeDtypeStruct((1, H), jnp.float32),
    ),
)


def _tc_pool_body(agg_ref, ndst_ref, b_ref, r_ref):
    ga, gb = _gcn_epilogue(agg_ref, ndst_ref, b_ref)

    @pl.when(pl.program_id(0) == 0)
    def _():
        r_ref[...] = jnp.zeros((1, H), jnp.float32)

    r_ref[0:1, :DH] += jnp.sum(ga, axis=0, keepdims=True) * (1.0 / N)
    r_ref[0:1, DH:] += jnp.sum(gb, axis=0, keepdims=True) * (1.0 / N)


_tc_pool = pl.pallas_call(
    _tc_pool_body,
    grid=(G_TC,),
    in_specs=[
        pl.BlockSpec((NC, BR, DH), lambda i: (0, i, 0)),
        pl.BlockSpec((BR, 1), lambda i: (i, 0)),
        pl.BlockSpec((1, H), lambda i: (0, 0)),
    ],
    out_specs=pl.BlockSpec((1, H), lambda i: (0, 0)),
    out_shape=jax.ShapeDtypeStruct((1, H), jnp.float32),
)


def _tc_head_body(r1_ref, r2_ref, r3_ref,
                  wf1_ref, bf1_ref, wf2_ref, bf2_ref, out_ref):
    t = (_dot(r1_ref[...], wf1_ref[0:H, :])
         + _dot(r2_ref[...], wf1_ref[H:2 * H, :])
         + _dot(r3_ref[...], wf1_ref[2 * H:, :]))
    fc1 = jnp.maximum(t + bf1_ref[...], 0.0)
    fc2 = _dot(fc1, wf2_ref[...]) + bf2_ref[...]
    out_ref[...] = jax.nn.sigmoid(fc2)


_tc_head = pl.pallas_call(
    _tc_head_body,
    out_shape=jax.ShapeDtypeStruct((1, 2), jnp.float32),
)


def kernel(x, edge_index, W1, b1, W2, b2, W3, b3, Wf1, bf1, Wf2, bf2):
    src = edge_index[0].astype(jnp.int32)
    dst = edge_index[1].astype(jnp.int32)

    npad = E_PAD - E
    srcb = jnp.concatenate([src, jnp.full((npad,), N - 1, jnp.int32)])
    srcb = srcb.reshape(CHUNKS_PAD, CHUNK)
    dstb = jnp.concatenate([dst, jnp.full((npad,), N, jnp.int32)])
    dstb = dstb.reshape(CHUNKS_PAD, CHUNK)

    deg2 = _sc_deg(src, dst)
    dsrc = deg2[0, :, 0:1]
    ddst = deg2[1, :, 0:1]

    xs, nsrc, ndst = _tc_prep(x, dsrc, ddst)
    agg1 = _sc_agg1(srcb, dstb, xs)
    ha, hb, r1 = _tc_l1(agg1, W1, b1.reshape(1, H), ndst, nsrc, W2)
    agg2 = _sc_agg(srcb, dstb, ha, hb)
    ha, hb, r2 = _tc_mid(agg2, ndst, b2.reshape(1, H), W3, nsrc)
    agg3 = _sc_agg(srcb, dstb, ha, hb)
    r3 = _tc_pool(agg3, ndst, b3.reshape(1, H))
    return _tc_head(r1, r2, r3,
                    Wf1, bf1.reshape(1, 128), Wf2, bf2.reshape(1, 2))


# speedup vs baseline: 2.0003x; 2.0003x over previous
import jax
import jax.numpy as jnp
from jax import lax
from jax.experimental import pallas as pl
from jax.experimental.pallas import tpu as pltpu
from jax.experimental.pallas import tpu_sc as plsc

N = 10000
E = 320000
D_IN = 128
H = 256
DH = H // 2

NC = 2
NS = 16
CHUNK = 128
N_CHUNKS = E // CHUNK
CHUNKS_PER_TILE = -(-N_CHUNKS // NS)
ZROWS = 125
N_ZSPANS = N // ZROWS
OUT_SPAN = 640
OUT_SPAN_LAST = N - 15 * OUT_SPAN
DEG_OUT_ROWS = 1256

_sc_mesh = plsc.VectorSubcoreMesh(core_axis_name="c", subcore_axis_name="s")


def _fill(buf, nrows, ncols, val):
    @pl.loop(0, nrows)
    def _(r):
        @pl.loop(0, ncols // 16)
        def _(c):
            buf[r, pl.ds(c * 16, 16)] = jnp.full((16,), val, jnp.float32)


def _zero_fill(buf, nrows, ncols):
    _fill(buf, nrows, ncols, 0.0)


def _sc_deg_body(src_hbm, dst_hbm, deg_hbm, idx0_v, idx1_v, ones_v, zeros_v,
                 sem0, sem1, deg_sh):
    cid = lax.axis_index("c")
    sid = lax.axis_index("s")

    _fill(ones_v, CHUNK, 128, 1.0)
    _zero_fill(zeros_v, ZROWS, 128)

    @pl.loop(0, N_ZSPANS)
    def _(k):
        @pl.when(lax.rem(k, NS) == sid)
        def _():
            pltpu.sync_copy(zeros_v, deg_sh.at[pl.ds(k * ZROWS, ZROWS)])
    plsc.subcore_barrier()

    def count(e_hbm):
        idx = (idx0_v, idx1_v)
        sems = (sem0, sem1)

        def start(c, b):
            pltpu.sync_copy(e_hbm.at[pl.ds(c * CHUNK, CHUNK)], idx[b])
            pltpu.async_copy(ones_v, deg_sh.at[idx[b]], sems[b], add=True)

        def drain(b):
            pltpu.make_async_copy(ones_v, deg_sh.at[idx[b]], sems[b]).wait()

        start(sid, 0)

        @pl.loop(0, CHUNKS_PER_TILE, step=2)
        def _(k):
            c_a = k * NS + sid
            c_b = c_a + NS
            c_c = c_b + NS

            @pl.when(c_b < N_CHUNKS)
            def _():
                start(c_b, 1)

            @pl.when(c_a < N_CHUNKS)
            def _():
                drain(0)

            @pl.when(c_c < N_CHUNKS)
            def _():
                start(c_c, 0)

            @pl.when(c_b < N_CHUNKS)
            def _():
                drain(1)

    @pl.when(cid == 0)
    def _():
        count(src_hbm)

    @pl.when(cid == 1)
    def _():
        count(dst_hbm)

    plsc.subcore_barrier()
    rbase = sid * OUT_SPAN

    @pl.when(sid < NS - 1)
    def _():
        pltpu.sync_copy(deg_sh.at[pl.ds(rbase, OUT_SPAN)],
                        deg_hbm.at[cid, pl.ds(rbase, OUT_SPAN)])

    @pl.when(sid == NS - 1)
    def _():
        pltpu.sync_copy(deg_sh.at[pl.ds(rbase, OUT_SPAN_LAST)],
                        deg_hbm.at[cid, pl.ds(rbase, OUT_SPAN_LAST)])


_sc_deg = pl.kernel(
    _sc_deg_body,
    out_type=jax.ShapeDtypeStruct((NC, N, 128), jnp.float32),
    mesh=_sc_mesh,
    scratch_types=[
        pltpu.VMEM((CHUNK,), jnp.int32),
        pltpu.VMEM((CHUNK,), jnp.int32),
        pltpu.VMEM((CHUNK, 128), jnp.float32),
        pltpu.VMEM((ZROWS, 128), jnp.float32),
        pltpu.SemaphoreType.DMA,
        pltpu.SemaphoreType.DMA,
        pltpu.VMEM_SHARED((N, 128), jnp.float32),
    ],
)


def _zero_spmem_acc(zrows, agg_sh, sid):
    _zero_fill(zrows, ZROWS, DH)

    @pl.loop(0, N_ZSPANS)
    def _(k):
        @pl.when(lax.rem(k, NS) == sid)
        def _():
            pltpu.sync_copy(zrows.at[pl.ds(0, ZROWS)],
                            agg_sh.at[pl.ds(k * ZROWS, ZROWS)])
    plsc.subcore_barrier()


def _edge_pipeline(src_hbm, dst_hbm, h_hbm, agg_sh, bufs, c0, lim, nslots):
    sidx, didx, rows, gsem = bufs

    def prep(c, b):
        base = c * CHUNK
        pltpu.sync_copy(src_hbm.at[pl.ds(base, CHUNK)], sidx[b])
        pltpu.sync_copy(dst_hbm.at[pl.ds(base, CHUNK)], didx[b])
        pltpu.async_copy(h_hbm.at[sidx[b]], rows[b], gsem[b])

    prep(c0, 0)
    prep(c0 + NS, 1)

    nslots3 = -(-nslots // 3) * 3

    @pl.loop(0, nslots3, step=3)
    def _(k):
        for j in range(3):
            b = j
            b2 = (j + 2) % 3
            slot = k + j
            c = c0 + slot * NS
            c2 = c + 2 * NS

            @pl.when(c2 < lim)
            def _():
                prep(c2, b2)

            @pl.when(c < lim)
            def _():
                pltpu.make_async_copy(h_hbm.at[sidx[b]], rows[b],
                                      gsem[b]).wait()
                pltpu.sync_copy(rows[b], agg_sh.at[didx[b]], add=True)


def _acc_copy_out(agg_sh, agg_hbm, cid, sid):
    plsc.subcore_barrier()
    rbase = sid * OUT_SPAN

    @pl.when(sid < NS - 1)
    def _():
        pltpu.sync_copy(agg_sh.at[pl.ds(rbase, OUT_SPAN)],
                        agg_hbm.at[cid, pl.ds(rbase, OUT_SPAN)])

    @pl.when(sid == NS - 1)
    def _():
        pltpu.sync_copy(agg_sh.at[pl.ds(rbase, OUT_SPAN_LAST)],
                        agg_hbm.at[cid, pl.ds(rbase, OUT_SPAN_LAST)])


def _sc_agg_body(src_hbm, dst_hbm, ha_hbm, hb_hbm, agg_hbm,
                 sidx0, sidx1, sidx2, didx0, didx1, didx2,
                 rows0, rows1, rows2, gsem0, gsem1, gsem2, agg_sh):
    cid = lax.axis_index("c")
    sid = lax.axis_index("s")
    _zero_spmem_acc(rows0, agg_sh, sid)

    bufs = ((sidx0, sidx1, sidx2), (didx0, didx1, didx2),
            (rows0, rows1, rows2), (gsem0, gsem1, gsem2))

    @pl.when(cid == 0)
    def _():
        _edge_pipeline(src_hbm, dst_hbm, ha_hbm, agg_sh, bufs,
                       sid, N_CHUNKS, CHUNKS_PER_TILE)

    @pl.when(cid == 1)
    def _():
        _edge_pipeline(src_hbm, dst_hbm, hb_hbm, agg_sh, bufs,
                       sid, N_CHUNKS, CHUNKS_PER_TILE)

    _acc_copy_out(agg_sh, agg_hbm, cid, sid)


_AGG_SCRATCH = (
    [pltpu.VMEM((CHUNK,), jnp.int32)] * 6
    + [pltpu.VMEM((CHUNK, DH), jnp.float32)] * 3
    + [pltpu.SemaphoreType.DMA] * 3
    + [pltpu.VMEM_SHARED((N, DH), jnp.float32)]
)

_sc_agg = pl.kernel(
    _sc_agg_body,
    out_type=jax.ShapeDtypeStruct((NC, N, DH), jnp.float32),
    mesh=_sc_mesh,
    scratch_types=list(_AGG_SCRATCH),
)

HALF_CHUNKS = N_CHUNKS // NC
CHUNKS_PER_TILE_H = -(-HALF_CHUNKS // NS)


def _sc_agg1_body(src_hbm, dst_hbm, xs_hbm, agg_hbm,
                  sidx0, sidx1, sidx2, didx0, didx1, didx2,
                  rows0, rows1, rows2, gsem0, gsem1, gsem2, agg_sh):
    cid = lax.axis_index("c")
    sid = lax.axis_index("s")
    _zero_spmem_acc(rows0, agg_sh, sid)

    bufs = ((sidx0, sidx1, sidx2), (didx0, didx1, didx2),
            (rows0, rows1, rows2), (gsem0, gsem1, gsem2))
    _edge_pipeline(src_hbm, dst_hbm, xs_hbm, agg_sh, bufs,
                   cid * HALF_CHUNKS + sid, (cid + 1) * HALF_CHUNKS,
                   CHUNKS_PER_TILE_H)

    _acc_copy_out(agg_sh, agg_hbm, cid, sid)


_sc_agg1 = pl.kernel(
    _sc_agg1_body,
    out_type=jax.ShapeDtypeStruct((NC, N, DH), jnp.float32),
    mesh=_sc_mesh,
    scratch_types=list(_AGG_SCRATCH),
)


def _dot(a, b):
    return jnp.dot(a, b, preferred_element_type=jnp.float32,
                   precision=lax.Precision.HIGHEST)


BR = 2000
G_TC = N // BR


def _tc_prep_body(x_ref, dsrc_ref, ddst_ref, xs_ref, nsrc_ref, ndst_ref):
    nsrc = lax.rsqrt(jnp.maximum(dsrc_ref[...], 1.0))
    ndst = lax.rsqrt(jnp.maximum(ddst_ref[...], 1.0))
    nsrc_ref[...] = nsrc
    ndst_ref[...] = ndst
    xs_ref[...] = x_ref[...] * nsrc


_tc_prep = pl.pallas_call(
    _tc_prep_body,
    grid=(G_TC,),
    in_specs=[
        pl.BlockSpec((BR, D_IN), lambda i: (i, 0)),
        pl.BlockSpec((BR, 1), lambda i: (i, 0)),
        pl.BlockSpec((BR, 1), lambda i: (i, 0)),
    ],
    out_specs=(
        pl.BlockSpec((BR, D_IN), lambda i: (i, 0)),
        pl.BlockSpec((BR, 1), lambda i: (i, 0)),
        pl.BlockSpec((BR, 1), lambda i: (i, 0)),
    ),
    out_shape=(
        jax.ShapeDtypeStruct((N, D_IN), jnp.float32),
        jax.ShapeDtypeStruct((N, 1), jnp.float32),
        jax.ShapeDtypeStruct((N, 1), jnp.float32),
    ),
)


def _tc_l1_body(agg_ref, w1_ref, b1_ref, ndst_ref, nsrc_ref, w2_ref,
                ha_ref, hb_ref, r_ref):
    a1 = agg_ref[0] + agg_ref[1]
    g = jnp.maximum(_dot(a1, w1_ref[...]) * ndst_ref[...] + b1_ref[...], 0.0)

    @pl.when(pl.program_id(0) == 0)
    def _():
        r_ref[...] = jnp.zeros((1, H), jnp.float32)

    r_ref[...] += jnp.sum(g, axis=0, keepdims=True) * (1.0 / N)
    hn = _dot(g, w2_ref[...]) * nsrc_ref[...]
    ha_ref[...] = hn[:, :DH]
    hb_ref[...] = hn[:, DH:]


_tc_l1 = pl.pallas_call(
    _tc_l1_body,
    grid=(G_TC,),
    in_specs=[
        pl.BlockSpec((NC, BR, DH), lambda i: (0, i, 0)),
        pl.BlockSpec((D_IN, H), lambda i: (0, 0)),
        pl.BlockSpec((1, H), lambda i: (0, 0)),
        pl.BlockSpec((BR, 1), lambda i: (i, 0)),
        pl.BlockSpec((BR, 1), lambda i: (i, 0)),
        pl.BlockSpec((H, H), lambda i: (0, 0)),
    ],
    out_specs=(
        pl.BlockSpec((BR, DH), lambda i: (i, 0)),
        pl.BlockSpec((BR, DH), lambda i: (i, 0)),
        pl.BlockSpec((1, H), lambda i: (0, 0)),
    ),
    out_shape=(
        jax.ShapeDtypeStruct((N, DH), jnp.float32),
        jax.ShapeDtypeStruct((N, DH), jnp.float32),
        jax.ShapeDtypeStruct((1, H), jnp.float32),
    ),
)


def _gcn_epilogue(agg_ref, ndst_ref, b_ref):
    ndst = ndst_ref[...]
    ga = jnp.maximum(agg_ref[0] * ndst + b_ref[0:1, :DH], 0.0)
    gb = jnp.maximum(agg_ref[1] * ndst + b_ref[0:1, DH:], 0.0)
    return ga, gb


def _tc_mid_body(agg_ref, ndst_ref, b_ref, w_ref, nsrc_ref,
                 ha_ref, hb_ref, r_ref):
    ga, gb = _gcn_epilogue(agg_ref, ndst_ref, b_ref)
    ra = jnp.sum(ga, axis=0, keepdims=True) * (1.0 / N)
    rb = jnp.sum(gb, axis=0, keepdims=True) * (1.0 / N)

    @pl.when(pl.program_id(0) == 0)
    def _():
        r_ref[...] = jnp.zeros((1, H), jnp.float32)

    r_ref[0:1, :DH] += ra
    r_ref[0:1, DH:] += rb
    hn = (_dot(ga, w_ref[:DH, :]) + _dot(gb, w_ref[DH:, :])) * nsrc_ref[...]
    ha_ref[...] = hn[:, :DH]
    hb_ref[...] = hn[:, DH:]


_tc_mid = pl.pallas_call(
    _tc_mid_body,
    grid=(G_TC,),
    in_specs=[
        pl.BlockSpec((NC, BR, DH), lambda i: (0, i, 0)),
        pl.BlockSpec((BR, 1), lambda i: (i, 0)),
        pl.BlockSpec((1, H), lambda i: (0, 0)),
        pl.BlockSpec((H, H), lambda i: (0, 0)),
        pl.BlockSpec((BR, 1), lambda i: (i, 0)),
    ],
    out_specs=(
        pl.BlockSpec((BR, DH), lambda i: (i, 0)),
        pl.BlockSpec((BR, DH), lambda i: (i, 0)),
        pl.BlockSpec((1, H), lambda i: (0, 0)),
    ),
    out_shape=(
        jax.ShapeDtypeStruct((N, DH), jnp.float32),
        jax.ShapeDtypeStruct((N, DH), jnp.float32),
        jax.ShapeDtypeStruct((1, H), jnp.float32),
    ),
)


def _tc_pool_body(agg_ref, ndst_ref, b_ref, r_ref):
    ga, gb = _gcn_epilogue(agg_ref, ndst_ref, b_ref)

    @pl.when(pl.program_id(0) == 0)
    def _():
        r_ref[...] = jnp.zeros((1, H), jnp.float32)

    r_ref[0:1, :DH] += jnp.sum(ga, axis=0, keepdims=True) * (1.0 / N)
    r_ref[0:1, DH:] += jnp.sum(gb, axis=0, keepdims=True) * (1.0 / N)


_tc_pool = pl.pallas_call(
    _tc_pool_body,
    grid=(G_TC,),
    in_specs=[
        pl.BlockSpec((NC, BR, DH), lambda i: (0, i, 0)),
        pl.BlockSpec((BR, 1), lambda i: (i, 0)),
        pl.BlockSpec((1, H), lambda i: (0, 0)),
    ],
    out_specs=pl.BlockSpec((1, H), lambda i: (0, 0)),
    out_shape=jax.ShapeDtypeStruct((1, H), jnp.float32),
)


def _tc_head_body(r1_ref, r2_ref, r3_ref,
                  wf1_ref, bf1_ref, wf2_ref, bf2_ref, out_ref):
    t = (_dot(r1_ref[...], wf1_ref[0:H, :])
         + _dot(r2_ref[...], wf1_ref[H:2 * H, :])
         + _dot(r3_ref[...], wf1_ref[2 * H:, :]))
    fc1 = jnp.maximum(t + bf1_ref[...], 0.0)
    fc2 = _dot(fc1, wf2_ref[...]) + bf2_ref[...]
    out_ref[...] = jax.nn.sigmoid(fc2)


_tc_head = pl.pallas_call(
    _tc_head_body,
    out_shape=jax.ShapeDtypeStruct((1, 2), jnp.float32),
)


def kernel(x, edge_index, W1, b1, W2, b2, W3, b3, Wf1, bf1, Wf2, bf2):
    src = edge_index[0].astype(jnp.int32)
    dst = edge_index[1].astype(jnp.int32)

    deg2 = _sc_deg(src, dst)
    dsrc = deg2[0, :, 0:1]
    ddst = deg2[1, :, 0:1]

    xs, nsrc, ndst = _tc_prep(x, dsrc, ddst)
    agg1 = _sc_agg1(src, dst, xs)
    ha, hb, r1 = _tc_l1(agg1, W1, b1.reshape(1, H), ndst, nsrc, W2)
    agg2 = _sc_agg(src, dst, ha, hb)
    ha, hb, r2 = _tc_mid(agg2, ndst, b2.reshape(1, H), W3, nsrc)
    agg3 = _sc_agg(src, dst, ha, hb)
    r3 = _tc_pool(agg3, ndst, b3.reshape(1, H))
    return _tc_head(r1, r2, r3,
                    Wf1, bf1.reshape(1, 128), Wf2, bf2.reshape(1, 2))


# speedup vs baseline: 2.5995x; 1.2996x over previous
import jax
import jax.numpy as jnp
from jax import lax
from jax.experimental import pallas as pl
from jax.experimental.pallas import tpu as pltpu
from jax.experimental.pallas import tpu_sc as plsc

N = 10000
E = 320000
D_IN = 128
H = 256
DH = H // 2

NC = 2
NS = 16
CHUNK = 128
N_CHUNKS = E // CHUNK
CHUNKS_PER_TILE = -(-N_CHUNKS // NS)
ZROWS = 125
N_ZSPANS = N // ZROWS
OUT_SPAN = 640
OUT_SPAN_LAST = N - 15 * OUT_SPAN
DEG_OUT_ROWS = 1256

_sc_mesh = plsc.VectorSubcoreMesh(core_axis_name="c", subcore_axis_name="s")


def _fill(buf, nrows, ncols, val):
    @pl.loop(0, nrows)
    def _(r):
        @pl.loop(0, ncols // 16)
        def _(c):
            buf[r, pl.ds(c * 16, 16)] = jnp.full((16,), val, jnp.float32)


def _zero_fill(buf, nrows, ncols):
    _fill(buf, nrows, ncols, 0.0)


def _sc_deg_body(src_hbm, dst_hbm, deg_hbm, idx0_v, idx1_v, ones_v, zeros_v,
                 sem0, sem1, deg_sh):
    cid = lax.axis_index("c")
    sid = lax.axis_index("s")

    _fill(ones_v, CHUNK, 128, 1.0)
    _zero_fill(zeros_v, ZROWS, 128)

    @pl.loop(0, N_ZSPANS)
    def _(k):
        @pl.when(lax.rem(k, NS) == sid)
        def _():
            pltpu.sync_copy(zeros_v, deg_sh.at[pl.ds(k * ZROWS, ZROWS)])
    plsc.subcore_barrier()

    def count(e_hbm):
        idx = (idx0_v, idx1_v)
        sems = (sem0, sem1)

        def start(c, b):
            pltpu.sync_copy(e_hbm.at[pl.ds(c * CHUNK, CHUNK)], idx[b])
            pltpu.async_copy(ones_v, deg_sh.at[idx[b]], sems[b], add=True)

        def drain(b):
            pltpu.make_async_copy(ones_v, deg_sh.at[idx[b]], sems[b]).wait()

        start(sid, 0)

        @pl.loop(0, CHUNKS_PER_TILE, step=2)
        def _(k):
            c_a = k * NS + sid
            c_b = c_a + NS
            c_c = c_b + NS

            @pl.when(c_b < N_CHUNKS)
            def _():
                start(c_b, 1)

            @pl.when(c_a < N_CHUNKS)
            def _():
                drain(0)

            @pl.when(c_c < N_CHUNKS)
            def _():
                start(c_c, 0)

            @pl.when(c_b < N_CHUNKS)
            def _():
                drain(1)

    @pl.when(cid == 0)
    def _():
        count(src_hbm)

    @pl.when(cid == 1)
    def _():
        count(dst_hbm)

    plsc.subcore_barrier()
    rbase = sid * OUT_SPAN

    @pl.when(sid < NS - 1)
    def _():
        pltpu.sync_copy(deg_sh.at[pl.ds(rbase, OUT_SPAN)],
                        deg_hbm.at[cid, pl.ds(rbase, OUT_SPAN)])

    @pl.when(sid == NS - 1)
    def _():
        pltpu.sync_copy(deg_sh.at[pl.ds(rbase, OUT_SPAN_LAST)],
                        deg_hbm.at[cid, pl.ds(rbase, OUT_SPAN_LAST)])


_sc_deg = pl.kernel(
    _sc_deg_body,
    out_type=jax.ShapeDtypeStruct((NC, N, 128), jnp.float32),
    mesh=_sc_mesh,
    scratch_types=[
        pltpu.VMEM((CHUNK,), jnp.int32),
        pltpu.VMEM((CHUNK,), jnp.int32),
        pltpu.VMEM((CHUNK, 128), jnp.float32),
        pltpu.VMEM((ZROWS, 128), jnp.float32),
        pltpu.SemaphoreType.DMA,
        pltpu.SemaphoreType.DMA,
        pltpu.VMEM_SHARED((N, 128), jnp.float32),
    ],
)


def _zero_spmem_acc(zrows, agg_sh, sid):
    _zero_fill(zrows, ZROWS, DH)

    @pl.loop(0, N_ZSPANS)
    def _(k):
        @pl.when(lax.rem(k, NS) == sid)
        def _():
            pltpu.sync_copy(zrows.at[pl.ds(0, ZROWS)],
                            agg_sh.at[pl.ds(k * ZROWS, ZROWS)])
    plsc.subcore_barrier()


def _edge_pipeline(src_hbm, dst_hbm, h_hbm, agg_sh, bufs, c0, lim, nslots):
    sidx, didx, rows, gsem, isem = bufs

    def idx_fire(c, b):
        base = c * CHUNK
        pltpu.async_copy(src_hbm.at[pl.ds(base, CHUNK)], sidx[b], isem[b])
        pltpu.async_copy(dst_hbm.at[pl.ds(base, CHUNK)], didx[b], isem[b])

    def idx_wait(c, b):
        base = c * CHUNK
        pltpu.make_async_copy(src_hbm.at[pl.ds(base, CHUNK)], sidx[b],
                              isem[b]).wait()
        pltpu.make_async_copy(dst_hbm.at[pl.ds(base, CHUNK)], didx[b],
                              isem[b]).wait()

    def idx_sync(c, b):
        base = c * CHUNK
        pltpu.sync_copy(src_hbm.at[pl.ds(base, CHUNK)], sidx[b])
        pltpu.sync_copy(dst_hbm.at[pl.ds(base, CHUNK)], didx[b])

    def gather_fire(b):
        pltpu.async_copy(h_hbm.at[sidx[b]], rows[b], gsem[b])

    idx_sync(c0, 0)
    gather_fire(0)
    idx_sync(c0 + NS, 1)
    gather_fire(1)
    idx_fire(c0 + 2 * NS, 2)

    nslots3 = -(-nslots // 3) * 3

    @pl.loop(0, nslots3, step=3)
    def _(k):
        for j in range(3):
            b = j
            b2 = (j + 2) % 3
            slot = k + j
            c = c0 + slot * NS
            c2 = c + 2 * NS
            c3 = c + 3 * NS

            @pl.when(c < lim)
            def _():
                pltpu.make_async_copy(h_hbm.at[sidx[b]], rows[b],
                                      gsem[b]).wait()
                pltpu.sync_copy(rows[b], agg_sh.at[didx[b]], add=True)

            @pl.when(c3 < lim)
            def _():
                idx_fire(c3, b)

            @pl.when(c2 < lim)
            def _():
                idx_wait(c2, b2)
                gather_fire(b2)


def _acc_copy_out(agg_sh, agg_hbm, cid, sid):
    plsc.subcore_barrier()
    rbase = sid * OUT_SPAN

    @pl.when(sid < NS - 1)
    def _():
        pltpu.sync_copy(agg_sh.at[pl.ds(rbase, OUT_SPAN)],
                        agg_hbm.at[cid, pl.ds(rbase, OUT_SPAN)])

    @pl.when(sid == NS - 1)
    def _():
        pltpu.sync_copy(agg_sh.at[pl.ds(rbase, OUT_SPAN_LAST)],
                        agg_hbm.at[cid, pl.ds(rbase, OUT_SPAN_LAST)])


def _sc_agg_body(src_hbm, dst_hbm, ha_hbm, hb_hbm, agg_hbm,
                 sidx0, sidx1, sidx2, didx0, didx1, didx2,
                 rows0, rows1, rows2, gsem0, gsem1, gsem2,
                 isem0, isem1, isem2, agg_sh):
    cid = lax.axis_index("c")
    sid = lax.axis_index("s")
    _zero_spmem_acc(rows0, agg_sh, sid)

    bufs = ((sidx0, sidx1, sidx2), (didx0, didx1, didx2),
            (rows0, rows1, rows2), (gsem0, gsem1, gsem2),
            (isem0, isem1, isem2))

    @pl.when(cid == 0)
    def _():
        _edge_pipeline(src_hbm, dst_hbm, ha_hbm, agg_sh, bufs,
                       sid, N_CHUNKS, CHUNKS_PER_TILE)

    @pl.when(cid == 1)
    def _():
        _edge_pipeline(src_hbm, dst_hbm, hb_hbm, agg_sh, bufs,
                       sid, N_CHUNKS, CHUNKS_PER_TILE)

    _acc_copy_out(agg_sh, agg_hbm, cid, sid)


_AGG_SCRATCH = (
    [pltpu.VMEM((CHUNK,), jnp.int32)] * 6
    + [pltpu.VMEM((CHUNK, DH), jnp.float32)] * 3
    + [pltpu.SemaphoreType.DMA] * 6
    + [pltpu.VMEM_SHARED((N, DH), jnp.float32)]
)

_sc_agg = pl.kernel(
    _sc_agg_body,
    out_type=jax.ShapeDtypeStruct((NC, N, DH), jnp.float32),
    mesh=_sc_mesh,
    scratch_types=list(_AGG_SCRATCH),
)

HALF_CHUNKS = N_CHUNKS // NC
CHUNKS_PER_TILE_H = -(-HALF_CHUNKS // NS)


def _sc_agg1_body(src_hbm, dst_hbm, xs_hbm, agg_hbm,
                  sidx0, sidx1, sidx2, didx0, didx1, didx2,
                  rows0, rows1, rows2, gsem0, gsem1, gsem2,
                  isem0, isem1, isem2, agg_sh):
    cid = lax.axis_index("c")
    sid = lax.axis_index("s")
    _zero_spmem_acc(rows0, agg_sh, sid)

    bufs = ((sidx0, sidx1, sidx2), (didx0, didx1, didx2),
            (rows0, rows1, rows2), (gsem0, gsem1, gsem2),
            (isem0, isem1, isem2))
    _edge_pipeline(src_hbm, dst_hbm, xs_hbm, agg_sh, bufs,
                   cid * HALF_CHUNKS + sid, (cid + 1) * HALF_CHUNKS,
                   CHUNKS_PER_TILE_H)

    _acc_copy_out(agg_sh, agg_hbm, cid, sid)


_sc_agg1 = pl.kernel(
    _sc_agg1_body,
    out_type=jax.ShapeDtypeStruct((NC, N, DH), jnp.float32),
    mesh=_sc_mesh,
    scratch_types=list(_AGG_SCRATCH),
)


def _dot(a, b):
    return jnp.dot(a, b, preferred_element_type=jnp.float32,
                   precision=lax.Precision.HIGHEST)


BR = 2000
G_TC = N // BR


def _tc_prep_body(x_ref, dsrc_ref, ddst_ref, xs_ref, nsrc_ref, ndst_ref):
    nsrc = lax.rsqrt(jnp.maximum(dsrc_ref[...], 1.0))
    ndst = lax.rsqrt(jnp.maximum(ddst_ref[...], 1.0))
    nsrc_ref[...] = nsrc
    ndst_ref[...] = ndst
    xs_ref[...] = x_ref[...] * nsrc


_tc_prep = pl.pallas_call(
    _tc_prep_body,
    grid=(G_TC,),
    in_specs=[
        pl.BlockSpec((BR, D_IN), lambda i: (i, 0)),
        pl.BlockSpec((BR, 1), lambda i: (i, 0)),
        pl.BlockSpec((BR, 1), lambda i: (i, 0)),
    ],
    out_specs=(
        pl.BlockSpec((BR, D_IN), lambda i: (i, 0)),
        pl.BlockSpec((BR, 1), lambda i: (i, 0)),
        pl.BlockSpec((BR, 1), lambda i: (i, 0)),
    ),
    out_shape=(
        jax.ShapeDtypeStruct((N, D_IN), jnp.float32),
        jax.ShapeDtypeStruct((N, 1), jnp.float32),
        jax.ShapeDtypeStruct((N, 1), jnp.float32),
    ),
)


def _tc_l1_body(agg_ref, w1_ref, b1_ref, ndst_ref, nsrc_ref, w2_ref,
                ha_ref, hb_ref, r_ref):
    a1 = agg_ref[0] + agg_ref[1]
    g = jnp.maximum(_dot(a1, w1_ref[...]) * ndst_ref[...] + b1_ref[...], 0.0)

    @pl.when(pl.program_id(0) == 0)
    def _():
        r_ref[...] = jnp.zeros((1, H), jnp.float32)

    r_ref[...] += jnp.sum(g, axis=0, keepdims=True) * (1.0 / N)
    hn = _dot(g, w2_ref[...]) * nsrc_ref[...]
    ha_ref[...] = hn[:, :DH]
    hb_ref[...] = hn[:, DH:]


_tc_l1 = pl.pallas_call(
    _tc_l1_body,
    grid=(G_TC,),
    in_specs=[
        pl.BlockSpec((NC, BR, DH), lambda i: (0, i, 0)),
        pl.BlockSpec((D_IN, H), lambda i: (0, 0)),
        pl.BlockSpec((1, H), lambda i: (0, 0)),
        pl.BlockSpec((BR, 1), lambda i: (i, 0)),
        pl.BlockSpec((BR, 1), lambda i: (i, 0)),
        pl.BlockSpec((H, H), lambda i: (0, 0)),
    ],
    out_specs=(
        pl.BlockSpec((BR, DH), lambda i: (i, 0)),
        pl.BlockSpec((BR, DH), lambda i: (i, 0)),
        pl.BlockSpec((1, H), lambda i: (0, 0)),
    ),
    out_shape=(
        jax.ShapeDtypeStruct((N, DH), jnp.float32),
        jax.ShapeDtypeStruct((N, DH), jnp.float32),
        jax.ShapeDtypeStruct((1, H), jnp.float32),
    ),
)


def _gcn_epilogue(agg_ref, ndst_ref, b_ref):
    ndst = ndst_ref[...]
    ga = jnp.maximum(agg_ref[0] * ndst + b_ref[0:1, :DH], 0.0)
    gb = jnp.maximum(agg_ref[1] * ndst + b_ref[0:1, DH:], 0.0)
    return ga, gb


def _tc_mid_body(agg_ref, ndst_ref, b_ref, w_ref, nsrc_ref,
                 ha_ref, hb_ref, r_ref):
    ga, gb = _gcn_epilogue(agg_ref, ndst_ref, b_ref)
    ra = jnp.sum(ga, axis=0, keepdims=True) * (1.0 / N)
    rb = jnp.sum(gb, axis=0, keepdims=True) * (1.0 / N)

    @pl.when(pl.program_id(0) == 0)
    def _():
        r_ref[...] = jnp.zeros((1, H), jnp.float32)

    r_ref[0:1, :DH] += ra
    r_ref[0:1, DH:] += rb
    hn = (_dot(ga, w_ref[:DH, :]) + _dot(gb, w_ref[DH:, :])) * nsrc_ref[...]
    ha_ref[...] = hn[:, :DH]
    hb_ref[...] = hn[:, DH:]


_tc_mid = pl.pallas_call(
    _tc_mid_body,
    grid=(G_TC,),
    in_specs=[
        pl.BlockSpec((NC, BR, DH), lambda i: (0, i, 0)),
        pl.BlockSpec((BR, 1), lambda i: (i, 0)),
        pl.BlockSpec((1, H), lambda i: (0, 0)),
        pl.BlockSpec((H, H), lambda i: (0, 0)),
        pl.BlockSpec((BR, 1), lambda i: (i, 0)),
    ],
    out_specs=(
        pl.BlockSpec((BR, DH), lambda i: (i, 0)),
        pl.BlockSpec((BR, DH), lambda i: (i, 0)),
        pl.BlockSpec((1, H), lambda i: (0, 0)),
    ),
    out_shape=(
        jax.ShapeDtypeStruct((N, DH), jnp.float32),
        jax.ShapeDtypeStruct((N, DH), jnp.float32),
        jax.ShapeDtypeStruct((1, H), jnp.float32),
    ),
)


def _tc_pool_body(agg_ref, ndst_ref, b_ref, r_ref):
    ga, gb = _gcn_epilogue(agg_ref, ndst_ref, b_ref)

    @pl.when(pl.program_id(0) == 0)
    def _():
        r_ref[...] = jnp.zeros((1, H), jnp.float32)

    r_ref[0:1, :DH] += jnp.sum(ga, axis=0, keepdims=True) * (1.0 / N)
    r_ref[0:1, DH:] += jnp.sum(gb, axis=0, keepdims=True) * (1.0 / N)


_tc_pool = pl.pallas_call(
    _tc_pool_body,
    grid=(G_TC,),
    in_specs=[
        pl.BlockSpec((NC, BR, DH), lambda i: (0, i, 0)),
        pl.BlockSpec((BR, 1), lambda i: (i, 0)),
        pl.BlockSpec((1, H), lambda i: (0, 0)),
    ],
    out_specs=pl.BlockSpec((1, H), lambda i: (0, 0)),
    out_shape=jax.ShapeDtypeStruct((1, H), jnp.float32),
)


def _tc_head_body(r1_ref, r2_ref, r3_ref,
                  wf1_ref, bf1_ref, wf2_ref, bf2_ref, out_ref):
    t = (_dot(r1_ref[...], wf1_ref[0:H, :])
         + _dot(r2_ref[...], wf1_ref[H:2 * H, :])
         + _dot(r3_ref[...], wf1_ref[2 * H:, :]))
    fc1 = jnp.maximum(t + bf1_ref[...], 0.0)
    fc2 = _dot(fc1, wf2_ref[...]) + bf2_ref[...]
    out_ref[...] = jax.nn.sigmoid(fc2)


_tc_head = pl.pallas_call(
    _tc_head_body,
    out_shape=jax.ShapeDtypeStruct((1, 2), jnp.float32),
)


def kernel(x, edge_index, W1, b1, W2, b2, W3, b3, Wf1, bf1, Wf2, bf2):
    src = edge_index[0].astype(jnp.int32)
    dst = edge_index[1].astype(jnp.int32)

    deg2 = _sc_deg(src, dst)
    dsrc = deg2[0, :, 0:1]
    ddst = deg2[1, :, 0:1]

    xs, nsrc, ndst = _tc_prep(x, dsrc, ddst)
    agg1 = _sc_agg1(src, dst, xs)
    ha, hb, r1 = _tc_l1(agg1, W1, b1.reshape(1, H), ndst, nsrc, W2)
    agg2 = _sc_agg(src, dst, ha, hb)
    ha, hb, r2 = _tc_mid(agg2, ndst, b2.reshape(1, H), W3, nsrc)
    agg3 = _sc_agg(src, dst, ha, hb)
    r3 = _tc_pool(agg3, ndst, b3.reshape(1, H))
    return _tc_head(r1, r2, r3,
                    Wf1, bf1.reshape(1, 128), Wf2, bf2.reshape(1, 2))


# speedup vs baseline: 2.6212x; 1.0084x over previous
import jax
import jax.numpy as jnp
from jax import lax
from jax.experimental import pallas as pl
from jax.experimental.pallas import tpu as pltpu
from jax.experimental.pallas import tpu_sc as plsc

N = 10000
E = 320000
D_IN = 128
H = 256
DH = H // 2

NC = 2
NS = 16
CHUNK = 128
N_CHUNKS = E // CHUNK
CHUNKS_PER_TILE = -(-N_CHUNKS // NS)
ZROWS = 125
N_ZSPANS = N // ZROWS
OUT_SPAN = 640
OUT_SPAN_LAST = N - 15 * OUT_SPAN
DEG_OUT_ROWS = 1256

_sc_mesh = plsc.VectorSubcoreMesh(core_axis_name="c", subcore_axis_name="s")


def _fill(buf, nrows, ncols, val):
    @pl.loop(0, nrows)
    def _(r):
        @pl.loop(0, ncols // 16)
        def _(c):
            buf[r, pl.ds(c * 16, 16)] = jnp.full((16,), val, jnp.float32)


def _zero_fill(buf, nrows, ncols):
    _fill(buf, nrows, ncols, 0.0)


def _sc_deg_body(src_hbm, dst_hbm, deg_hbm, idx0_v, idx1_v, idx2_v,
                 ones_v, zeros_v, isem0, isem1, isem2,
                 ssem0, ssem1, ssem2, deg_sh):
    cid = lax.axis_index("c")
    sid = lax.axis_index("s")

    _fill(ones_v, CHUNK, 128, 1.0)
    _zero_fill(zeros_v, ZROWS, 128)

    @pl.loop(0, N_ZSPANS)
    def _(k):
        @pl.when(lax.rem(k, NS) == sid)
        def _():
            pltpu.sync_copy(zeros_v, deg_sh.at[pl.ds(k * ZROWS, ZROWS)])
    plsc.subcore_barrier()

    def count(e_hbm):
        idx = (idx0_v, idx1_v, idx2_v)
        isems = (isem0, isem1, isem2)
        ssems = (ssem0, ssem1, ssem2)

        def idx_fire(c, b):
            pltpu.async_copy(e_hbm.at[pl.ds(c * CHUNK, CHUNK)], idx[b],
                             isems[b])

        def idx_wait(c, b):
            pltpu.make_async_copy(e_hbm.at[pl.ds(c * CHUNK, CHUNK)], idx[b],
                                  isems[b]).wait()

        idx_fire(sid, 0)
        idx_fire(sid + NS, 1)
        idx_fire(sid + 2 * NS, 2)

        nslots3 = -(-CHUNKS_PER_TILE // 3) * 3

        @pl.loop(0, nslots3 + 3, step=3)
        def _(k):
            for j in range(3):
                b = j
                bp = (j + 2) % 3
                slot = k + j
                c = sid + slot * NS
                c2 = c + 2 * NS

                @pl.when(c < N_CHUNKS)
                def _():
                    idx_wait(c, b)
                    pltpu.async_copy(ones_v, deg_sh.at[idx[b]], ssems[b],
                                     add=True)

                @pl.when((slot >= 1) & (c - NS < N_CHUNKS))
                def _():
                    pltpu.make_async_copy(ones_v, deg_sh.at[idx[bp]],
                                          ssems[bp]).wait()

                @pl.when(c2 < N_CHUNKS)
                def _():
                    idx_fire(c2, bp)

    @pl.when(cid == 0)
    def _():
        count(src_hbm)

    @pl.when(cid == 1)
    def _():
        count(dst_hbm)

    plsc.subcore_barrier()
    rbase = sid * OUT_SPAN

    @pl.when(sid < NS - 1)
    def _():
        pltpu.sync_copy(deg_sh.at[pl.ds(rbase, OUT_SPAN)],
                        deg_hbm.at[cid, pl.ds(rbase, OUT_SPAN)])

    @pl.when(sid == NS - 1)
    def _():
        pltpu.sync_copy(deg_sh.at[pl.ds(rbase, OUT_SPAN_LAST)],
                        deg_hbm.at[cid, pl.ds(rbase, OUT_SPAN_LAST)])


_sc_deg = pl.kernel(
    _sc_deg_body,
    out_type=jax.ShapeDtypeStruct((NC, N, 128), jnp.float32),
    mesh=_sc_mesh,
    scratch_types=(
        [pltpu.VMEM((CHUNK,), jnp.int32)] * 3
        + [pltpu.VMEM((CHUNK, 128), jnp.float32),
           pltpu.VMEM((ZROWS, 128), jnp.float32)]
        + [pltpu.SemaphoreType.DMA] * 6
        + [pltpu.VMEM_SHARED((N, 128), jnp.float32)]
    ),
)


def _zero_spmem_acc(zrows, agg_sh, sid):
    _zero_fill(zrows, ZROWS, DH)

    @pl.loop(0, N_ZSPANS)
    def _(k):
        @pl.when(lax.rem(k, NS) == sid)
        def _():
            pltpu.sync_copy(zrows.at[pl.ds(0, ZROWS)],
                            agg_sh.at[pl.ds(k * ZROWS, ZROWS)])
    plsc.subcore_barrier()


def _edge_pipeline(src_hbm, dst_hbm, h_hbm, agg_sh, bufs, c0, lim, nslots):
    sidx, didx, rows, gsem, isem = bufs

    def idx_fire(c, b):
        base = c * CHUNK
        pltpu.async_copy(src_hbm.at[pl.ds(base, CHUNK)], sidx[b], isem[b])
        pltpu.async_copy(dst_hbm.at[pl.ds(base, CHUNK)], didx[b], isem[b])

    def idx_wait(c, b):
        base = c * CHUNK
        pltpu.make_async_copy(src_hbm.at[pl.ds(base, CHUNK)], sidx[b],
                              isem[b]).wait()
        pltpu.make_async_copy(dst_hbm.at[pl.ds(base, CHUNK)], didx[b],
                              isem[b]).wait()

    def idx_sync(c, b):
        base = c * CHUNK
        pltpu.sync_copy(src_hbm.at[pl.ds(base, CHUNK)], sidx[b])
        pltpu.sync_copy(dst_hbm.at[pl.ds(base, CHUNK)], didx[b])

    def gather_fire(b):
        pltpu.async_copy(h_hbm.at[sidx[b]], rows[b], gsem[b])

    idx_sync(c0, 0)
    gather_fire(0)
    idx_sync(c0 + NS, 1)
    gather_fire(1)
    idx_fire(c0 + 2 * NS, 2)

    nslots3 = -(-nslots // 3) * 3

    @pl.loop(0, nslots3, step=3)
    def _(k):
        for j in range(3):
            b = j
            b2 = (j + 2) % 3
            slot = k + j
            c = c0 + slot * NS
            c2 = c + 2 * NS
            c3 = c + 3 * NS

            @pl.when(c < lim)
            def _():
                pltpu.make_async_copy(h_hbm.at[sidx[b]], rows[b],
                                      gsem[b]).wait()
                pltpu.sync_copy(rows[b], agg_sh.at[didx[b]], add=True)

            @pl.when(c3 < lim)
            def _():
                idx_fire(c3, b)

            @pl.when(c2 < lim)
            def _():
                idx_wait(c2, b2)
                gather_fire(b2)


def _acc_copy_out(agg_sh, agg_hbm, cid, sid):
    plsc.subcore_barrier()
    rbase = sid * OUT_SPAN

    @pl.when(sid < NS - 1)
    def _():
        pltpu.sync_copy(agg_sh.at[pl.ds(rbase, OUT_SPAN)],
                        agg_hbm.at[cid, pl.ds(rbase, OUT_SPAN)])

    @pl.when(sid == NS - 1)
    def _():
        pltpu.sync_copy(agg_sh.at[pl.ds(rbase, OUT_SPAN_LAST)],
                        agg_hbm.at[cid, pl.ds(rbase, OUT_SPAN_LAST)])


def _sc_agg_body(src_hbm, dst_hbm, ha_hbm, hb_hbm, agg_hbm,
                 sidx0, sidx1, sidx2, didx0, didx1, didx2,
                 rows0, rows1, rows2, gsem0, gsem1, gsem2,
                 isem0, isem1, isem2, agg_sh):
    cid = lax.axis_index("c")
    sid = lax.axis_index("s")
    _zero_spmem_acc(rows0, agg_sh, sid)

    bufs = ((sidx0, sidx1, sidx2), (didx0, didx1, didx2),
            (rows0, rows1, rows2), (gsem0, gsem1, gsem2),
            (isem0, isem1, isem2))

    @pl.when(cid == 0)
    def _():
        _edge_pipeline(src_hbm, dst_hbm, ha_hbm, agg_sh, bufs,
                       sid, N_CHUNKS, CHUNKS_PER_TILE)

    @pl.when(cid == 1)
    def _():
        _edge_pipeline(src_hbm, dst_hbm, hb_hbm, agg_sh, bufs,
                       sid, N_CHUNKS, CHUNKS_PER_TILE)

    _acc_copy_out(agg_sh, agg_hbm, cid, sid)


_AGG_SCRATCH = (
    [pltpu.VMEM((CHUNK,), jnp.int32)] * 6
    + [pltpu.VMEM((CHUNK, DH), jnp.float32)] * 3
    + [pltpu.SemaphoreType.DMA] * 6
    + [pltpu.VMEM_SHARED((N, DH), jnp.float32)]
)

_sc_agg = pl.kernel(
    _sc_agg_body,
    out_type=jax.ShapeDtypeStruct((NC, N, DH), jnp.float32),
    mesh=_sc_mesh,
    scratch_types=list(_AGG_SCRATCH),
)

HALF_CHUNKS = N_CHUNKS // NC
CHUNKS_PER_TILE_H = -(-HALF_CHUNKS // NS)


def _sc_agg1_body(src_hbm, dst_hbm, xs_hbm, agg_hbm,
                  sidx0, sidx1, sidx2, didx0, didx1, didx2,
                  rows0, rows1, rows2, gsem0, gsem1, gsem2,
                  isem0, isem1, isem2, agg_sh):
    cid = lax.axis_index("c")
    sid = lax.axis_index("s")
    _zero_spmem_acc(rows0, agg_sh, sid)

    bufs = ((sidx0, sidx1, sidx2), (didx0, didx1, didx2),
            (rows0, rows1, rows2), (gsem0, gsem1, gsem2),
            (isem0, isem1, isem2))
    _edge_pipeline(src_hbm, dst_hbm, xs_hbm, agg_sh, bufs,
                   cid * HALF_CHUNKS + sid, (cid + 1) * HALF_CHUNKS,
                   CHUNKS_PER_TILE_H)

    _acc_copy_out(agg_sh, agg_hbm, cid, sid)


_sc_agg1 = pl.kernel(
    _sc_agg1_body,
    out_type=jax.ShapeDtypeStruct((NC, N, DH), jnp.float32),
    mesh=_sc_mesh,
    scratch_types=list(_AGG_SCRATCH),
)


def _dot(a, b):
    return jnp.dot(a, b, preferred_element_type=jnp.float32,
                   precision=lax.Precision.HIGHEST)


BR = 2000
G_TC = N // BR


def _tc_prep_body(x_ref, dsrc_ref, ddst_ref, xs_ref, nsrc_ref, ndst_ref):
    nsrc = lax.rsqrt(jnp.maximum(dsrc_ref[...], 1.0))
    ndst = lax.rsqrt(jnp.maximum(ddst_ref[...], 1.0))
    nsrc_ref[...] = nsrc
    ndst_ref[...] = ndst
    xs_ref[...] = x_ref[...] * nsrc


_tc_prep = pl.pallas_call(
    _tc_prep_body,
    grid=(G_TC,),
    in_specs=[
        pl.BlockSpec((BR, D_IN), lambda i: (i, 0)),
        pl.BlockSpec((BR, 1), lambda i: (i, 0)),
        pl.BlockSpec((BR, 1), lambda i: (i, 0)),
    ],
    out_specs=(
        pl.BlockSpec((BR, D_IN), lambda i: (i, 0)),
        pl.BlockSpec((BR, 1), lambda i: (i, 0)),
        pl.BlockSpec((BR, 1), lambda i: (i, 0)),
    ),
    out_shape=(
        jax.ShapeDtypeStruct((N, D_IN), jnp.float32),
        jax.ShapeDtypeStruct((N, 1), jnp.float32),
        jax.ShapeDtypeStruct((N, 1), jnp.float32),
    ),
)


def _tc_l1_body(agg_ref, w1_ref, b1_ref, ndst_ref, nsrc_ref, w2_ref,
                ha_ref, hb_ref, r_ref):
    a1 = agg_ref[0] + agg_ref[1]
    g = jnp.maximum(_dot(a1, w1_ref[...]) * ndst_ref[...] + b1_ref[...], 0.0)

    @pl.when(pl.program_id(0) == 0)
    def _():
        r_ref[...] = jnp.zeros((1, H), jnp.float32)

    r_ref[...] += jnp.sum(g, axis=0, keepdims=True) * (1.0 / N)
    hn = _dot(g, w2_ref[...]) * nsrc_ref[...]
    ha_ref[...] = hn[:, :DH]
    hb_ref[...] = hn[:, DH:]


_tc_l1 = pl.pallas_call(
    _tc_l1_body,
    grid=(G_TC,),
    in_specs=[
        pl.BlockSpec((NC, BR, DH), lambda i: (0, i, 0)),
        pl.BlockSpec((D_IN, H), lambda i: (0, 0)),
        pl.BlockSpec((1, H), lambda i: (0, 0)),
        pl.BlockSpec((BR, 1), lambda i: (i, 0)),
        pl.BlockSpec((BR, 1), lambda i: (i, 0)),
        pl.BlockSpec((H, H), lambda i: (0, 0)),
    ],
    out_specs=(
        pl.BlockSpec((BR, DH), lambda i: (i, 0)),
        pl.BlockSpec((BR, DH), lambda i: (i, 0)),
        pl.BlockSpec((1, H), lambda i: (0, 0)),
    ),
    out_shape=(
        jax.ShapeDtypeStruct((N, DH), jnp.float32),
        jax.ShapeDtypeStruct((N, DH), jnp.float32),
        jax.ShapeDtypeStruct((1, H), jnp.float32),
    ),
)


def _gcn_epilogue(agg_ref, ndst_ref, b_ref):
    ndst = ndst_ref[...]
    ga = jnp.maximum(agg_ref[0] * ndst + b_ref[0:1, :DH], 0.0)
    gb = jnp.maximum(agg_ref[1] * ndst + b_ref[0:1, DH:], 0.0)
    return ga, gb


def _tc_mid_body(agg_ref, ndst_ref, b_ref, w_ref, nsrc_ref,
                 ha_ref, hb_ref, r_ref):
    ga, gb = _gcn_epilogue(agg_ref, ndst_ref, b_ref)
    ra = jnp.sum(ga, axis=0, keepdims=True) * (1.0 / N)
    rb = jnp.sum(gb, axis=0, keepdims=True) * (1.0 / N)

    @pl.when(pl.program_id(0) == 0)
    def _():
        r_ref[...] = jnp.zeros((1, H), jnp.float32)

    r_ref[0:1, :DH] += ra
    r_ref[0:1, DH:] += rb
    hn = (_dot(ga, w_ref[:DH, :]) + _dot(gb, w_ref[DH:, :])) * nsrc_ref[...]
    ha_ref[...] = hn[:, :DH]
    hb_ref[...] = hn[:, DH:]


_tc_mid = pl.pallas_call(
    _tc_mid_body,
    grid=(G_TC,),
    in_specs=[
        pl.BlockSpec((NC, BR, DH), lambda i: (0, i, 0)),
        pl.BlockSpec((BR, 1), lambda i: (i, 0)),
        pl.BlockSpec((1, H), lambda i: (0, 0)),
        pl.BlockSpec((H, H), lambda i: (0, 0)),
        pl.BlockSpec((BR, 1), lambda i: (i, 0)),
    ],
    out_specs=(
        pl.BlockSpec((BR, DH), lambda i: (i, 0)),
        pl.BlockSpec((BR, DH), lambda i: (i, 0)),
        pl.BlockSpec((1, H), lambda i: (0, 0)),
    ),
    out_shape=(
        jax.ShapeDtypeStruct((N, DH), jnp.float32),
        jax.ShapeDtypeStruct((N, DH), jnp.float32),
        jax.ShapeDtypeStruct((1, H), jnp.float32),
    ),
)


def _tc_pool_body(agg_ref, ndst_ref, b_ref, r_ref):
    ga, gb = _gcn_epilogue(agg_ref, ndst_ref, b_ref)

    @pl.when(pl.program_id(0) == 0)
    def _():
        r_ref[...] = jnp.zeros((1, H), jnp.float32)

    r_ref[0:1, :DH] += jnp.sum(ga, axis=0, keepdims=True) * (1.0 / N)
    r_ref[0:1, DH:] += jnp.sum(gb, axis=0, keepdims=True) * (1.0 / N)


_tc_pool = pl.pallas_call(
    _tc_pool_body,
    grid=(G_TC,),
    in_specs=[
        pl.BlockSpec((NC, BR, DH), lambda i: (0, i, 0)),
        pl.BlockSpec((BR, 1), lambda i: (i, 0)),
        pl.BlockSpec((1, H), lambda i: (0, 0)),
    ],
    out_specs=pl.BlockSpec((1, H), lambda i: (0, 0)),
    out_shape=jax.ShapeDtypeStruct((1, H), jnp.float32),
)


def _tc_head_body(r1_ref, r2_ref, r3_ref,
                  wf1_ref, bf1_ref, wf2_ref, bf2_ref, out_ref):
    t = (_dot(r1_ref[...], wf1_ref[0:H, :])
         + _dot(r2_ref[...], wf1_ref[H:2 * H, :])
         + _dot(r3_ref[...], wf1_ref[2 * H:, :]))
    fc1 = jnp.maximum(t + bf1_ref[...], 0.0)
    fc2 = _dot(fc1, wf2_ref[...]) + bf2_ref[...]
    out_ref[...] = jax.nn.sigmoid(fc2)


_tc_head = pl.pallas_call(
    _tc_head_body,
    out_shape=jax.ShapeDtypeStruct((1, 2), jnp.float32),
)


def kernel(x, edge_index, W1, b1, W2, b2, W3, b3, Wf1, bf1, Wf2, bf2):
    src = edge_index[0].astype(jnp.int32)
    dst = edge_index[1].astype(jnp.int32)

    deg2 = _sc_deg(src, dst)
    dsrc = deg2[0, :, 0:1]
    ddst = deg2[1, :, 0:1]

    xs, nsrc, ndst = _tc_prep(x, dsrc, ddst)
    agg1 = _sc_agg1(src, dst, xs)
    ha, hb, r1 = _tc_l1(agg1, W1, b1.reshape(1, H), ndst, nsrc, W2)
    agg2 = _sc_agg(src, dst, ha, hb)
    ha, hb, r2 = _tc_mid(agg2, ndst, b2.reshape(1, H), W3, nsrc)
    agg3 = _sc_agg(src, dst, ha, hb)
    r3 = _tc_pool(agg3, ndst, b3.reshape(1, H))
    return _tc_head(r1, r2, r3,
                    Wf1, bf1.reshape(1, 128), Wf2, bf2.reshape(1, 2))
